# baseline (speedup 1.0000x reference)
import functools
import math

import jax
import jax.numpy as jnp
from jax import lax
from jax.experimental import pallas as pl
from jax.experimental.pallas import tpu as pltpu
from jax.experimental.pallas import tpu_sc as plsc

B, S, D, V = 4096, 200, 64, 1000000
SCALE = math.sqrt(float(D))

NC, NS, L = 2, 16, 16
NW = NC * NS
ROWS_W = (B * S) // NW
CHUNK = 128
NCH = ROWS_W // CHUNK
NBUF = 2
PE_EXT = S + CHUNK


def _sc_embed(x3, table, pos_enc):
    mesh = plsc.VectorSubcoreMesh(core_axis_name="c", subcore_axis_name="s")

    @functools.partial(
        pl.kernel,
        mesh=mesh,
        out_type=jax.ShapeDtypeStruct((B * S, D), jnp.float32),
        compiler_params=pltpu.CompilerParams(use_tc_tiling_on_sc=False),
        scratch_types=[
            pltpu.VMEM((NCH, CHUNK), jnp.int32),
            pltpu.VMEM((NBUF, CHUNK, D), jnp.float32),
            pltpu.VMEM((PE_EXT, D), jnp.float32),
            pltpu.SemaphoreType.DMA,
            pltpu.SemaphoreType.DMA,
        ],
    )
    def k(x_hbm, table_hbm, pe_hbm, out_hbm, idx_v, rows_v, pe_v, sem0, sem1):
        sems = (sem0, sem1)
        wid = lax.axis_index("s") * NC + lax.axis_index("c")
        base = wid * ROWS_W

        pltpu.sync_copy(x_hbm.at[wid], idx_v)
        pltpu.sync_copy(pe_hbm, pe_v.at[pl.ds(0, S)])
        pltpu.sync_copy(pe_hbm.at[pl.ds(0, CHUNK)], pe_v.at[pl.ds(S, CHUNK)])

        def gather(b, t):
            return pltpu.make_async_copy(
                table_hbm.at[idx_v.at[t]], rows_v.at[b], sems[b])

        for b in range(NBUF):
            gather(b, b).start()

        def outer(jj, carry):
            for b in range(NBUF):
                t = jj * NBUF + b
                gather(b, t).wait()
                s0 = lax.rem(t * CHUNK, S)

                def row(r, c_):
                    pr = s0 + r
                    for c in range(D // L):
                        sl = pl.ds(c * L, L)
                        rows_v[b, r, sl] = rows_v[b, r, sl] * SCALE + pe_v[pr, sl]
                    return c_

                lax.fori_loop(0, CHUNK, row, 0)
                pltpu.sync_copy(
                    rows_v.at[b], out_hbm.at[pl.ds(base + t * CHUNK, CHUNK)])

                @pl.when(t + NBUF < NCH)
                def _():
                    gather(b, t + NBUF).start()
            return carry

        lax.fori_loop(0, NCH // NBUF, outer, 0)

    return k(x3, table, pos_enc)


def kernel(x, table, pos_enc):
    x3 = x.astype(jnp.int32).reshape(NW, NCH, CHUNK)
    out = _sc_embed(x3, table, pos_enc)
    return out.reshape(B, S, D)


# speedup vs baseline: 1.3003x; 1.3003x over previous
import functools
import math

import jax
import jax.numpy as jnp
from jax import lax
from jax.experimental import pallas as pl
from jax.experimental.pallas import tpu as pltpu
from jax.experimental.pallas import tpu_sc as plsc

B, S, D, V = 4096, 200, 64, 1000000
SCALE = math.sqrt(float(D))

NC, NS, L = 2, 16, 16
NW = NC * NS
ROWS_W = (B * S) // NW
CHUNK = 128
NCH = ROWS_W // CHUNK
NBUF = 2
PE_EXT = S + CHUNK


def _sc_embed(x3, table, pos_enc):
    mesh = plsc.VectorSubcoreMesh(core_axis_name="c", subcore_axis_name="s")

    @functools.partial(
        pl.kernel,
        mesh=mesh,
        out_type=jax.ShapeDtypeStruct((B * S, D), jnp.float32),
        compiler_params=pltpu.CompilerParams(use_tc_tiling_on_sc=False),
        scratch_types=[
            pltpu.VMEM((NCH, CHUNK), jnp.int32),
            pltpu.VMEM((NBUF, CHUNK, D), jnp.float32),
            pltpu.VMEM((PE_EXT, D), jnp.float32),
            pltpu.SemaphoreType.DMA,
            pltpu.SemaphoreType.DMA,
        ],
    )
    def k(x_hbm, table_hbm, pe_hbm, out_hbm, idx_v, rows_v, pe_v, sem0, sem1):
        sems = (sem0, sem1)
        wid = lax.axis_index("s") * NC + lax.axis_index("c")
        base = wid * ROWS_W

        pltpu.sync_copy(x_hbm.at[wid], idx_v)
        pltpu.sync_copy(pe_hbm, pe_v.at[pl.ds(0, S)])
        pltpu.sync_copy(pe_hbm.at[pl.ds(0, CHUNK)], pe_v.at[pl.ds(S, CHUNK)])

        def gather(b, t):
            return pltpu.make_async_copy(
                table_hbm.at[idx_v.at[t]], rows_v.at[b], sems[b])

        for b in range(NBUF):
            gather(b, b).start()

        def outer(jj, carry):
            for b in range(NBUF):
                t = jj * NBUF + b
                gather(b, t).wait()
                s0 = lax.rem(t * CHUNK, S)

                @plsc.parallel_loop(0, CHUNK, unroll=8)
                def row(r):
                    pr = s0 + r
                    for c in range(D // L):
                        sl = pl.ds(c * L, L)
                        rows_v[b, r, sl] = rows_v[b, r, sl] * SCALE + pe_v[pr, sl]
                pltpu.sync_copy(
                    rows_v.at[b], out_hbm.at[pl.ds(base + t * CHUNK, CHUNK)])

                @pl.when(t + NBUF < NCH)
                def _():
                    gather(b, t + NBUF).start()
            return carry

        lax.fori_loop(0, NCH // NBUF, outer, 0)

    return k(x3, table, pos_enc)


def kernel(x, table, pos_enc):
    x3 = x.astype(jnp.int32).reshape(NW, NCH, CHUNK)
    out = _sc_embed(x3, table, pos_enc)
    return out.reshape(B, S, D)


# speedup vs baseline: 1.3263x; 1.0200x over previous
import functools
import math

import jax
import jax.numpy as jnp
from jax import lax
from jax.experimental import pallas as pl
from jax.experimental.pallas import tpu as pltpu
from jax.experimental.pallas import tpu_sc as plsc

B, S, D, V = 4096, 200, 64, 1000000
SCALE = math.sqrt(float(D))

NC, NS, L = 2, 16, 16
NW = NC * NS
ROWS_W = (B * S) // NW
CHUNK = 128
NCH = ROWS_W // CHUNK
NBUF = 4
PE_EXT = S + CHUNK


def _sc_embed(x3, table, pos_enc):
    mesh = plsc.VectorSubcoreMesh(core_axis_name="c", subcore_axis_name="s")

    @functools.partial(
        pl.kernel,
        mesh=mesh,
        out_type=jax.ShapeDtypeStruct((B * S, D), jnp.float32),
        compiler_params=pltpu.CompilerParams(use_tc_tiling_on_sc=False),
        scratch_types=[
            pltpu.VMEM((NCH, CHUNK), jnp.int32),
            pltpu.VMEM((NBUF, CHUNK, D), jnp.float32),
            pltpu.VMEM((PE_EXT, D), jnp.float32),
            [pltpu.SemaphoreType.DMA] * NBUF,
            [pltpu.SemaphoreType.DMA] * NBUF,
        ],
    )
    def k(x_hbm, table_hbm, pe_hbm, out_hbm, idx_v, rows_v, pe_v, gsems, wsems):
        wid = lax.axis_index("s") * NC + lax.axis_index("c")
        base = wid * ROWS_W

        pltpu.sync_copy(x_hbm.at[wid], idx_v)
        pltpu.sync_copy(pe_hbm, pe_v.at[pl.ds(0, S)])
        pltpu.sync_copy(pe_hbm.at[pl.ds(0, CHUNK)], pe_v.at[pl.ds(S, CHUNK)])

        def gather(b, t):
            return pltpu.make_async_copy(
                table_hbm.at[idx_v.at[t]], rows_v.at[b], gsems[b])

        def write(b, t):
            return pltpu.make_async_copy(
                rows_v.at[b], out_hbm.at[pl.ds(base + t * CHUNK, CHUNK)],
                wsems[b])

        for b in range(NBUF):
            gather(b, b).start()

        def outer(jj, carry):
            for b in range(NBUF):
                t = jj * NBUF + b
                bp = (b - 1) % NBUF

                @pl.when((t > 0) & (t + NBUF - 1 < NCH))
                def _():
                    write(bp, t - 1).wait()
                    gather(bp, t + NBUF - 1).start()

                gather(b, t).wait()
                s0 = lax.rem(t * CHUNK, S)

                @plsc.parallel_loop(0, CHUNK, unroll=8)
                def row(r):
                    pr = s0 + r
                    for c in range(D // L):
                        sl = pl.ds(c * L, L)
                        rows_v[b, r, sl] = rows_v[b, r, sl] * SCALE + pe_v[pr, sl]

                write(b, t).start()
            return carry

        lax.fori_loop(0, NCH // NBUF, outer, 0)

        for b in range(NBUF):
            write(b, NCH - NBUF + b).wait()

    return k(x3, table, pos_enc)


def kernel(x, table, pos_enc):
    x3 = x.astype(jnp.int32).reshape(NW, NCH, CHUNK)
    out = _sc_embed(x3, table, pos_enc)
    return out.reshape(B, S, D)
